# baseline (speedup 1.0000x reference)
import jax
import jax.numpy as jnp
from jax.experimental import pallas as pl
from jax.experimental.pallas import tpu as pltpu

TN1 = 1024
TN2 = 2048
BIG = 1e30
_COUNT = 4 * 8192.0


def _bf(x):
    return x.astype(jnp.bfloat16)


def _nn_interp_kernel(ut_ref, kn_ref, kf_ref, uf_ref, w0_ref,
                      y0_ref, st_ref):
    b = pl.program_id(0)
    t = pl.program_id(1)
    ut = ut_ref[...]
    kn = kn_ref[...]
    ux, uy, uz = ut[0:1, :], ut[1:2, :], ut[2:3, :]
    kx, ky, kz = kn[:, 0:1], kn[:, 1:2], kn[:, 2:3]
    dx = ux - kx
    dy = uy - ky
    dz = uz - kz
    d2 = (dx * dx + dy * dy) + dz * dz
    mn0 = jnp.min(d2, axis=0, keepdims=True)
    m0 = jnp.where(d2 == mn0, BIG, d2)
    mn1 = jnp.min(m0, axis=0, keepdims=True)
    mn2 = jnp.min(jnp.where(m0 == mn1, BIG, m0), axis=0, keepdims=True)
    r0 = 1.0 / (mn0 + 1e-8)
    r1 = 1.0 / (mn1 + 1e-8)
    r2 = 1.0 / (mn2 + 1e-8)
    norm = r0 + r1 + r2
    wt = jnp.where(d2 == mn0, r0 / norm,
                   jnp.where(d2 == mn1, r1 / norm,
                             jnp.where(d2 == mn2, r2 / norm, 0.0)))
    interp = jnp.dot(kf_ref[...], wt, preferred_element_type=jnp.float32,
                     precision=jax.lax.Precision.HIGHEST)
    w0 = w0_ref[...]
    y0 = (jnp.dot(_bf(w0[:, :64]), _bf(interp), preferred_element_type=jnp.float32)
          + jnp.dot(_bf(w0[:, 64:]), _bf(uf_ref[...]), preferred_element_type=jnp.float32))
    y0_ref[...] = y0

    @pl.when(jnp.logical_and(b == 0, t == 0))
    def _init():
        st_ref[...] = jnp.zeros_like(st_ref)

    s = jnp.sum(y0, axis=1, keepdims=True)
    q = jnp.sum(y0 * y0, axis=1, keepdims=True)
    st_ref[...] += jnp.concatenate([s, q], axis=1)


def _bn_mm_kernel(x_ref, sin_ref, w_ref, y_ref, st_ref):
    b = pl.program_id(0)
    t = pl.program_id(1)
    scale, shift = _scale_shift(sin_ref[...])
    a = jnp.maximum(x_ref[...] * scale + shift, 0.0)
    y = jnp.dot(_bf(w_ref[...]), _bf(a), preferred_element_type=jnp.float32)
    y_ref[...] = y

    @pl.when(jnp.logical_and(b == 0, t == 0))
    def _init():
        st_ref[...] = jnp.zeros_like(st_ref)

    s = jnp.sum(y, axis=1, keepdims=True)
    q = jnp.sum(y * y, axis=1, keepdims=True)
    st_ref[...] += jnp.concatenate([s, q], axis=1)


def _stage3_kernel(y1_ref, sin_ref, ut_ref, wpos_ref, wkey_ref, wv1_ref,
                   wq_ref, wk_ref, wattn_ref, h_ref, ap_ref, st_ref):
    b = pl.program_id(0)
    t = pl.program_id(1)
    scale, shift = _scale_shift(sin_ref[...])
    nf = jnp.maximum(y1_ref[...] * scale + shift, 0.0)
    keye = jnp.maximum(
        jnp.dot(_bf(wkey_ref[...]), _bf(nf), preferred_element_type=jnp.float32), 0.0)
    h = jnp.dot(_bf(wv1_ref[...]), _bf(nf), preferred_element_type=jnp.float32)
    h_ref[...] = h
    pos = jnp.maximum(
        jnp.dot(_bf(wpos_ref[...]), _bf(ut_ref[...]), preferred_element_type=jnp.float32), 0.0)
    q_c = jax.nn.sigmoid(
        jnp.dot(_bf(wq_ref[...]), _bf(pos), preferred_element_type=jnp.float32))
    k_c = jax.nn.sigmoid(
        jnp.dot(_bf(wk_ref[...]), _bf(keye), preferred_element_type=jnp.float32))
    emb = pos * q_c + keye * k_c
    ap_ref[...] = jnp.dot(_bf(wattn_ref[...]), _bf(emb),
                          preferred_element_type=jnp.float32)

    @pl.when(jnp.logical_and(b == 0, t == 0))
    def _init():
        st_ref[...] = jnp.zeros_like(st_ref)

    s = jnp.sum(h, axis=1, keepdims=True)
    q = jnp.sum(h * h, axis=1, keepdims=True)
    st_ref[...] += jnp.concatenate([s, q], axis=1)


def _final_kernel(vp_ref, sin_ref, ut_ref, wpos_ref, wvc_ref, ap_ref, out_ref):
    scale, shift = _scale_shift(sin_ref[...])
    val = jnp.maximum(vp_ref[...] * scale + shift, 0.0)
    pos = jnp.maximum(
        jnp.dot(_bf(wpos_ref[...]), _bf(ut_ref[...]), preferred_element_type=jnp.float32), 0.0)
    v_c = jax.nn.sigmoid(
        jnp.dot(_bf(wvc_ref[...]), _bf(pos), preferred_element_type=jnp.float32))
    val = val + pos * v_c
    ap = ap_ref[...]
    mx = jnp.max(ap, axis=1, keepdims=True)
    e = jnp.exp(ap - mx)
    asm = e / jnp.sum(e, axis=1, keepdims=True)
    ci = jax.lax.broadcasted_iota(jnp.int32, (64, 4), 0) // 16
    hj = jax.lax.broadcasted_iota(jnp.int32, (64, 4), 1)
    rep_mat = (ci == hj).astype(jnp.float32)
    rep = jnp.dot(rep_mat, asm, preferred_element_type=jnp.float32,
                  precision=jax.lax.Precision.HIGHEST)
    out_ref[...] = rep * val


def _scale_shift(st):
    mu = st[:, 0:1] * (1.0 / _COUNT)
    var = st[:, 1:2] * (1.0 / _COUNT) - mu * mu
    inv = jax.lax.rsqrt(var + 1e-5)
    return inv, -mu * inv


def kernel(unknown, known, unknow_feats, known_feats, mlp_w0, mlp_w1, w_pos,
           w_key, w_v1, w_v2, w_attn, w_q, w_k, w_qk, w_vc):
    B, n, _ = unknown.shape
    m = known.shape[1]
    C = 64
    ut = jnp.transpose(unknown, (0, 2, 1))

    g1 = (B, n // TN1)
    y0pre, st0 = pl.pallas_call(
        _nn_interp_kernel,
        grid=g1,
        in_specs=[
            pl.BlockSpec((None, 3, TN1), lambda b, t: (b, 0, t)),
            pl.BlockSpec((None, m, 3), lambda b, t: (b, 0, 0)),
            pl.BlockSpec((None, C, m), lambda b, t: (b, 0, 0)),
            pl.BlockSpec((None, C, TN1), lambda b, t: (b, 0, t)),
            pl.BlockSpec((C, 2 * C), lambda b, t: (0, 0)),
        ],
        out_specs=[
            pl.BlockSpec((None, C, TN1), lambda b, t: (b, 0, t)),
            pl.BlockSpec((C, 2), lambda b, t: (0, 0)),
        ],
        out_shape=[
            jax.ShapeDtypeStruct((B, C, n), jnp.float32),
            jax.ShapeDtypeStruct((C, 2), jnp.float32),
        ],
    )(ut, known, known_feats, unknow_feats, mlp_w0)

    g2 = (B, n // TN2)
    xspec = pl.BlockSpec((None, C, TN2), lambda b, t: (b, 0, t))
    stspec = pl.BlockSpec((C, 2), lambda b, t: (0, 0))

    y1pre, st1 = pl.pallas_call(
        _bn_mm_kernel,
        grid=g2,
        in_specs=[xspec, pl.BlockSpec((C, 2), lambda b, t: (0, 0)),
                  pl.BlockSpec((C, C), lambda b, t: (0, 0))],
        out_specs=[xspec, stspec],
        out_shape=[jax.ShapeDtypeStruct((B, C, n), jnp.float32),
                   jax.ShapeDtypeStruct((C, 2), jnp.float32)],
    )(y0pre, st0, mlp_w1)

    hpre, attnpre, st2 = pl.pallas_call(
        _stage3_kernel,
        grid=g2,
        in_specs=[xspec, pl.BlockSpec((C, 2), lambda b, t: (0, 0)),
                  pl.BlockSpec((None, 3, TN2), lambda b, t: (b, 0, t)),
                  pl.BlockSpec((C, 3), lambda b, t: (0, 0)),
                  pl.BlockSpec((C, C), lambda b, t: (0, 0)),
                  pl.BlockSpec((32, C), lambda b, t: (0, 0)),
                  pl.BlockSpec((1, C), lambda b, t: (0, 0)),
                  pl.BlockSpec((1, C), lambda b, t: (0, 0)),
                  pl.BlockSpec((4, C), lambda b, t: (0, 0))],
        out_specs=[pl.BlockSpec((None, 32, TN2), lambda b, t: (b, 0, t)),
                   pl.BlockSpec((None, 4, TN2), lambda b, t: (b, 0, t)),
                   pl.BlockSpec((32, 2), lambda b, t: (0, 0))],
        out_shape=[jax.ShapeDtypeStruct((B, 32, n), jnp.float32),
                   jax.ShapeDtypeStruct((B, 4, n), jnp.float32),
                   jax.ShapeDtypeStruct((32, 2), jnp.float32)],
    )(y1pre, st1, ut, w_pos, w_key, w_v1, w_q, w_k, w_attn)

    valpre, st3 = pl.pallas_call(
        _bn_mm_kernel,
        grid=g2,
        in_specs=[pl.BlockSpec((None, 32, TN2), lambda b, t: (b, 0, t)),
                  pl.BlockSpec((32, 2), lambda b, t: (0, 0)),
                  pl.BlockSpec((C, 32), lambda b, t: (0, 0))],
        out_specs=[xspec, stspec],
        out_shape=[jax.ShapeDtypeStruct((B, C, n), jnp.float32),
                   jax.ShapeDtypeStruct((C, 2), jnp.float32)],
    )(hpre, st2, w_v2)

    out = pl.pallas_call(
        _final_kernel,
        grid=(B,),
        in_specs=[pl.BlockSpec((None, C, n), lambda b: (b, 0, 0)),
                  pl.BlockSpec((C, 2), lambda b: (0, 0)),
                  pl.BlockSpec((None, 3, n), lambda b: (b, 0, 0)),
                  pl.BlockSpec((C, 3), lambda b: (0, 0)),
                  pl.BlockSpec((1, C), lambda b: (0, 0)),
                  pl.BlockSpec((None, 4, n), lambda b: (b, 0, 0))],
        out_specs=pl.BlockSpec((None, C, n), lambda b: (b, 0, 0)),
        out_shape=jax.ShapeDtypeStruct((B, C, n), jnp.float32),
    )(valpre, st3, ut, w_pos, w_vc, attnpre)
    return out


# speedup vs baseline: 1.0007x; 1.0007x over previous
import jax
import jax.numpy as jnp
from jax.experimental import pallas as pl

TN1 = 1024
TN2 = 2048
BIG = 1e30
_COUNT = 4 * 8192.0


def _bf(x):
    return x.astype(jnp.bfloat16)


def _nn_interp_kernel(ut_ref, kn_ref, kf_ref, uf_ref, w0_ref,
                      y0_ref, st_ref):
    b = pl.program_id(0)
    t = pl.program_id(1)
    ut = ut_ref[...]
    kn = kn_ref[...]
    ux, uy, uz = ut[0:1, :], ut[1:2, :], ut[2:3, :]
    kx, ky, kz = kn[:, 0:1], kn[:, 1:2], kn[:, 2:3]
    dx = ux - kx
    dy = uy - ky
    dz = uz - kz
    d2 = (dx * dx + dy * dy) + dz * dz
    mn0 = jnp.min(d2, axis=0, keepdims=True)
    m0 = jnp.where(d2 == mn0, BIG, d2)
    mn1 = jnp.min(m0, axis=0, keepdims=True)
    mn2 = jnp.min(jnp.where(m0 == mn1, BIG, m0), axis=0, keepdims=True)
    r0 = 1.0 / (mn0 + 1e-8)
    r1 = 1.0 / (mn1 + 1e-8)
    r2 = 1.0 / (mn2 + 1e-8)
    norm = r0 + r1 + r2
    wt = jnp.where(d2 == mn0, r0 / norm,
                   jnp.where(d2 == mn1, r1 / norm,
                             jnp.where(d2 == mn2, r2 / norm, 0.0)))
    interp = jnp.dot(kf_ref[...], wt, preferred_element_type=jnp.float32,
                     precision=jax.lax.Precision.HIGHEST)
    w0 = w0_ref[...]
    y0 = (jnp.dot(_bf(w0[:, :64]), _bf(interp), preferred_element_type=jnp.float32)
          + jnp.dot(_bf(w0[:, 64:]), _bf(uf_ref[...]), preferred_element_type=jnp.float32))
    y0_ref[...] = y0

    @pl.when(jnp.logical_and(b == 0, t == 0))
    def _init():
        st_ref[...] = jnp.zeros_like(st_ref)

    s = jnp.sum(y0, axis=1, keepdims=True)
    q = jnp.sum(y0 * y0, axis=1, keepdims=True)
    st_ref[...] += jnp.concatenate([s, q], axis=1)


def _bn_mm_kernel(x_ref, sin_ref, w_ref, y_ref, st_ref):
    b = pl.program_id(0)
    t = pl.program_id(1)
    scale, shift = _scale_shift(sin_ref[...])
    a = jnp.maximum(x_ref[...] * scale + shift, 0.0)
    y = jnp.dot(_bf(w_ref[...]), _bf(a), preferred_element_type=jnp.float32)
    y_ref[...] = y

    @pl.when(jnp.logical_and(b == 0, t == 0))
    def _init():
        st_ref[...] = jnp.zeros_like(st_ref)

    s = jnp.sum(y, axis=1, keepdims=True)
    q = jnp.sum(y * y, axis=1, keepdims=True)
    st_ref[...] += jnp.concatenate([s, q], axis=1)


def _stage3_kernel(y1_ref, sin_ref, ut_ref, wpos_ref, wkey_ref, wv1_ref,
                   wq_ref, wk_ref, wattn_ref, h_ref, ap_ref, st_ref):
    b = pl.program_id(0)
    t = pl.program_id(1)
    scale, shift = _scale_shift(sin_ref[...])
    nf = jnp.maximum(y1_ref[...] * scale + shift, 0.0)
    keye = jnp.maximum(
        jnp.dot(_bf(wkey_ref[...]), _bf(nf), preferred_element_type=jnp.float32), 0.0)
    h = jnp.dot(_bf(wv1_ref[...]), _bf(nf), preferred_element_type=jnp.float32)
    h_ref[...] = h
    pos = jnp.maximum(
        jnp.dot(_bf(wpos_ref[...]), _bf(ut_ref[...]), preferred_element_type=jnp.float32), 0.0)
    q_c = jax.nn.sigmoid(
        jnp.dot(_bf(wq_ref[...]), _bf(pos), preferred_element_type=jnp.float32))
    k_c = jax.nn.sigmoid(
        jnp.dot(_bf(wk_ref[...]), _bf(keye), preferred_element_type=jnp.float32))
    emb = pos * q_c + keye * k_c
    ap_ref[...] = jnp.dot(_bf(wattn_ref[...]), _bf(emb),
                          preferred_element_type=jnp.float32)

    @pl.when(jnp.logical_and(b == 0, t == 0))
    def _init():
        st_ref[...] = jnp.zeros_like(st_ref)

    s = jnp.sum(h, axis=1, keepdims=True)
    q = jnp.sum(h * h, axis=1, keepdims=True)
    st_ref[...] += jnp.concatenate([s, q], axis=1)


def _final_kernel(vp_ref, sin_ref, ut_ref, wpos_ref, wvc_ref, ap_ref, out_ref):
    scale, shift = _scale_shift(sin_ref[...])
    val = jnp.maximum(vp_ref[...] * scale + shift, 0.0)
    pos = jnp.maximum(
        jnp.dot(_bf(wpos_ref[...]), _bf(ut_ref[...]), preferred_element_type=jnp.float32), 0.0)
    v_c = jax.nn.sigmoid(
        jnp.dot(_bf(wvc_ref[...]), _bf(pos), preferred_element_type=jnp.float32))
    val = val + pos * v_c
    ap = ap_ref[...]
    mx = jnp.max(ap, axis=1, keepdims=True)
    e = jnp.exp(ap - mx)
    asm = e / jnp.sum(e, axis=1, keepdims=True)
    ci = jax.lax.broadcasted_iota(jnp.int32, (64, 4), 0) // 16
    hj = jax.lax.broadcasted_iota(jnp.int32, (64, 4), 1)
    rep_mat = (ci == hj).astype(jnp.float32)
    rep = jnp.dot(rep_mat, asm, preferred_element_type=jnp.float32,
                  precision=jax.lax.Precision.HIGHEST)
    out_ref[...] = rep * val


def _scale_shift(st):
    mu = st[:, 0:1] * (1.0 / _COUNT)
    var = st[:, 1:2] * (1.0 / _COUNT) - mu * mu
    inv = jax.lax.rsqrt(var + 1e-5)
    return inv, -mu * inv


def kernel(unknown, known, unknow_feats, known_feats, mlp_w0, mlp_w1, w_pos,
           w_key, w_v1, w_v2, w_attn, w_q, w_k, w_qk, w_vc):
    B, n, _ = unknown.shape
    m = known.shape[1]
    C = 64
    ut = jnp.transpose(unknown, (0, 2, 1))

    g1 = (B, n // TN1)
    y0pre, st0 = pl.pallas_call(
        _nn_interp_kernel,
        grid=g1,
        in_specs=[
            pl.BlockSpec((None, 3, TN1), lambda b, t: (b, 0, t)),
            pl.BlockSpec((None, m, 3), lambda b, t: (b, 0, 0)),
            pl.BlockSpec((None, C, m), lambda b, t: (b, 0, 0)),
            pl.BlockSpec((None, C, TN1), lambda b, t: (b, 0, t)),
            pl.BlockSpec((C, 2 * C), lambda b, t: (0, 0)),
        ],
        out_specs=[
            pl.BlockSpec((None, C, TN1), lambda b, t: (b, 0, t)),
            pl.BlockSpec((C, 2), lambda b, t: (0, 0)),
        ],
        out_shape=[
            jax.ShapeDtypeStruct((B, C, n), jnp.float32),
            jax.ShapeDtypeStruct((C, 2), jnp.float32),
        ],
    )(ut, known, known_feats, unknow_feats, mlp_w0)

    g2 = (B, n // TN2)
    xspec = pl.BlockSpec((None, C, TN2), lambda b, t: (b, 0, t))
    stspec = pl.BlockSpec((C, 2), lambda b, t: (0, 0))

    y1pre, st1 = pl.pallas_call(
        _bn_mm_kernel,
        grid=g2,
        in_specs=[xspec, pl.BlockSpec((C, 2), lambda b, t: (0, 0)),
                  pl.BlockSpec((C, C), lambda b, t: (0, 0))],
        out_specs=[xspec, stspec],
        out_shape=[jax.ShapeDtypeStruct((B, C, n), jnp.float32),
                   jax.ShapeDtypeStruct((C, 2), jnp.float32)],
    )(y0pre, st0, mlp_w1)

    hpre, attnpre, st2 = pl.pallas_call(
        _stage3_kernel,
        grid=g2,
        in_specs=[xspec, pl.BlockSpec((C, 2), lambda b, t: (0, 0)),
                  pl.BlockSpec((None, 3, TN2), lambda b, t: (b, 0, t)),
                  pl.BlockSpec((C, 3), lambda b, t: (0, 0)),
                  pl.BlockSpec((C, C), lambda b, t: (0, 0)),
                  pl.BlockSpec((32, C), lambda b, t: (0, 0)),
                  pl.BlockSpec((1, C), lambda b, t: (0, 0)),
                  pl.BlockSpec((1, C), lambda b, t: (0, 0)),
                  pl.BlockSpec((4, C), lambda b, t: (0, 0))],
        out_specs=[pl.BlockSpec((None, 32, TN2), lambda b, t: (b, 0, t)),
                   pl.BlockSpec((None, 4, TN2), lambda b, t: (b, 0, t)),
                   pl.BlockSpec((32, 2), lambda b, t: (0, 0))],
        out_shape=[jax.ShapeDtypeStruct((B, 32, n), jnp.float32),
                   jax.ShapeDtypeStruct((B, 4, n), jnp.float32),
                   jax.ShapeDtypeStruct((32, 2), jnp.float32)],
    )(y1pre, st1, ut, w_pos, w_key, w_v1, w_q, w_k, w_attn)

    valpre, st3 = pl.pallas_call(
        _bn_mm_kernel,
        grid=g2,
        in_specs=[pl.BlockSpec((None, 32, TN2), lambda b, t: (b, 0, t)),
                  pl.BlockSpec((32, 2), lambda b, t: (0, 0)),
                  pl.BlockSpec((C, 32), lambda b, t: (0, 0))],
        out_specs=[xspec, stspec],
        out_shape=[jax.ShapeDtypeStruct((B, C, n), jnp.float32),
                   jax.ShapeDtypeStruct((C, 2), jnp.float32)],
    )(hpre, st2, w_v2)

    out = pl.pallas_call(
        _final_kernel,
        grid=(B,),
        in_specs=[pl.BlockSpec((None, C, n), lambda b: (b, 0, 0)),
                  pl.BlockSpec((C, 2), lambda b: (0, 0)),
                  pl.BlockSpec((None, 3, n), lambda b: (b, 0, 0)),
                  pl.BlockSpec((C, 3), lambda b: (0, 0)),
                  pl.BlockSpec((1, C), lambda b: (0, 0)),
                  pl.BlockSpec((None, 4, n), lambda b: (b, 0, 0))],
        out_specs=pl.BlockSpec((None, C, n), lambda b: (b, 0, 0)),
        out_shape=jax.ShapeDtypeStruct((B, C, n), jnp.float32),
    )(valpre, st3, ut, w_pos, w_vc, attnpre)
    return out


# speedup vs baseline: 1.0627x; 1.0620x over previous
import jax
import jax.numpy as jnp
from jax.experimental import pallas as pl

TN1 = 1024
TN2 = 8192
BIG = 1e30
_COUNT = 4 * 8192.0


def _bf(x):
    return x.astype(jnp.bfloat16)


def _nn_interp_kernel(ut_ref, kn_ref, kf_ref, uf_ref, w0_ref,
                      y0_ref, st_ref):
    b = pl.program_id(0)
    t = pl.program_id(1)
    ut = ut_ref[...]
    kn = kn_ref[...]
    ux, uy, uz = ut[0:1, :], ut[1:2, :], ut[2:3, :]
    kx, ky, kz = kn[:, 0:1], kn[:, 1:2], kn[:, 2:3]
    dx = ux - kx
    dy = uy - ky
    dz = uz - kz
    d2 = (dx * dx + dy * dy) + dz * dz
    mn0 = jnp.min(d2, axis=0, keepdims=True)
    m0 = jnp.where(d2 == mn0, BIG, d2)
    mn1 = jnp.min(m0, axis=0, keepdims=True)
    mn2 = jnp.min(jnp.where(m0 == mn1, BIG, m0), axis=0, keepdims=True)
    r0 = 1.0 / (mn0 + 1e-8)
    r1 = 1.0 / (mn1 + 1e-8)
    r2 = 1.0 / (mn2 + 1e-8)
    norm = r0 + r1 + r2
    wt = jnp.where(d2 == mn0, r0 / norm,
                   jnp.where(d2 == mn1, r1 / norm,
                             jnp.where(d2 == mn2, r2 / norm, 0.0)))
    interp = jnp.dot(kf_ref[...], wt, preferred_element_type=jnp.float32,
                     precision=jax.lax.Precision.HIGHEST)
    w0 = w0_ref[...]
    y0 = (jnp.dot(_bf(w0[:, :64]), _bf(interp), preferred_element_type=jnp.float32)
          + jnp.dot(_bf(w0[:, 64:]), _bf(uf_ref[...]), preferred_element_type=jnp.float32))
    y0_ref[...] = y0

    @pl.when(jnp.logical_and(b == 0, t == 0))
    def _init():
        st_ref[...] = jnp.zeros_like(st_ref)

    s = jnp.sum(y0, axis=1, keepdims=True)
    q = jnp.sum(y0 * y0, axis=1, keepdims=True)
    st_ref[...] += jnp.concatenate([s, q], axis=1)


def _bn_mm_kernel(x_ref, sin_ref, w_ref, y_ref, st_ref):
    b = pl.program_id(0)
    t = pl.program_id(1)
    scale, shift = _scale_shift(sin_ref[...])
    a = jnp.maximum(x_ref[...] * scale + shift, 0.0)
    y = jnp.dot(_bf(w_ref[...]), _bf(a), preferred_element_type=jnp.float32)
    y_ref[...] = y

    @pl.when(jnp.logical_and(b == 0, t == 0))
    def _init():
        st_ref[...] = jnp.zeros_like(st_ref)

    s = jnp.sum(y, axis=1, keepdims=True)
    q = jnp.sum(y * y, axis=1, keepdims=True)
    st_ref[...] += jnp.concatenate([s, q], axis=1)


def _stage3_kernel(y1_ref, sin_ref, ut_ref, wpos_ref, wkey_ref, wv1_ref,
                   wq_ref, wk_ref, wattn_ref, h_ref, ap_ref, st_ref):
    b = pl.program_id(0)
    t = pl.program_id(1)
    scale, shift = _scale_shift(sin_ref[...])
    nf = jnp.maximum(y1_ref[...] * scale + shift, 0.0)
    keye = jnp.maximum(
        jnp.dot(_bf(wkey_ref[...]), _bf(nf), preferred_element_type=jnp.float32), 0.0)
    h = jnp.dot(_bf(wv1_ref[...]), _bf(nf), preferred_element_type=jnp.float32)
    h_ref[...] = h
    pos = jnp.maximum(
        jnp.dot(_bf(wpos_ref[...]), _bf(ut_ref[...]), preferred_element_type=jnp.float32), 0.0)
    q_c = jax.nn.sigmoid(
        jnp.dot(_bf(wq_ref[...]), _bf(pos), preferred_element_type=jnp.float32))
    k_c = jax.nn.sigmoid(
        jnp.dot(_bf(wk_ref[...]), _bf(keye), preferred_element_type=jnp.float32))
    emb = pos * q_c + keye * k_c
    ap_ref[...] = jnp.dot(_bf(wattn_ref[...]), _bf(emb),
                          preferred_element_type=jnp.float32)

    @pl.when(jnp.logical_and(b == 0, t == 0))
    def _init():
        st_ref[...] = jnp.zeros_like(st_ref)

    s = jnp.sum(h, axis=1, keepdims=True)
    q = jnp.sum(h * h, axis=1, keepdims=True)
    st_ref[...] += jnp.concatenate([s, q], axis=1)


def _final_kernel(vp_ref, sin_ref, ut_ref, wpos_ref, wvc_ref, ap_ref, out_ref):
    scale, shift = _scale_shift(sin_ref[...])
    val = jnp.maximum(vp_ref[...] * scale + shift, 0.0)
    pos = jnp.maximum(
        jnp.dot(_bf(wpos_ref[...]), _bf(ut_ref[...]), preferred_element_type=jnp.float32), 0.0)
    v_c = jax.nn.sigmoid(
        jnp.dot(_bf(wvc_ref[...]), _bf(pos), preferred_element_type=jnp.float32))
    val = val + pos * v_c
    ap = ap_ref[...]
    mx = jnp.max(ap, axis=1, keepdims=True)
    e = jnp.exp(ap - mx)
    asm = e / jnp.sum(e, axis=1, keepdims=True)
    ci = jax.lax.broadcasted_iota(jnp.int32, (64, 4), 0) // 16
    hj = jax.lax.broadcasted_iota(jnp.int32, (64, 4), 1)
    rep_mat = (ci == hj).astype(jnp.float32)
    rep = jnp.dot(rep_mat, asm, preferred_element_type=jnp.float32,
                  precision=jax.lax.Precision.HIGHEST)
    out_ref[...] = rep * val


def _scale_shift(st):
    mu = st[:, 0:1] * (1.0 / _COUNT)
    var = st[:, 1:2] * (1.0 / _COUNT) - mu * mu
    inv = jax.lax.rsqrt(var + 1e-5)
    return inv, -mu * inv


def kernel(unknown, known, unknow_feats, known_feats, mlp_w0, mlp_w1, w_pos,
           w_key, w_v1, w_v2, w_attn, w_q, w_k, w_qk, w_vc):
    B, n, _ = unknown.shape
    m = known.shape[1]
    C = 64
    ut = jnp.transpose(unknown, (0, 2, 1))

    g1 = (B, n // TN1)
    y0pre, st0 = pl.pallas_call(
        _nn_interp_kernel,
        grid=g1,
        in_specs=[
            pl.BlockSpec((None, 3, TN1), lambda b, t: (b, 0, t)),
            pl.BlockSpec((None, m, 3), lambda b, t: (b, 0, 0)),
            pl.BlockSpec((None, C, m), lambda b, t: (b, 0, 0)),
            pl.BlockSpec((None, C, TN1), lambda b, t: (b, 0, t)),
            pl.BlockSpec((C, 2 * C), lambda b, t: (0, 0)),
        ],
        out_specs=[
            pl.BlockSpec((None, C, TN1), lambda b, t: (b, 0, t)),
            pl.BlockSpec((C, 2), lambda b, t: (0, 0)),
        ],
        out_shape=[
            jax.ShapeDtypeStruct((B, C, n), jnp.float32),
            jax.ShapeDtypeStruct((C, 2), jnp.float32),
        ],
    )(ut, known, known_feats, unknow_feats, mlp_w0)

    g2 = (B, n // TN2)
    xspec = pl.BlockSpec((None, C, TN2), lambda b, t: (b, 0, t))
    stspec = pl.BlockSpec((C, 2), lambda b, t: (0, 0))

    y1pre, st1 = pl.pallas_call(
        _bn_mm_kernel,
        grid=g2,
        in_specs=[xspec, pl.BlockSpec((C, 2), lambda b, t: (0, 0)),
                  pl.BlockSpec((C, C), lambda b, t: (0, 0))],
        out_specs=[xspec, stspec],
        out_shape=[jax.ShapeDtypeStruct((B, C, n), jnp.float32),
                   jax.ShapeDtypeStruct((C, 2), jnp.float32)],
    )(y0pre, st0, mlp_w1)

    hpre, attnpre, st2 = pl.pallas_call(
        _stage3_kernel,
        grid=g2,
        in_specs=[xspec, pl.BlockSpec((C, 2), lambda b, t: (0, 0)),
                  pl.BlockSpec((None, 3, TN2), lambda b, t: (b, 0, t)),
                  pl.BlockSpec((C, 3), lambda b, t: (0, 0)),
                  pl.BlockSpec((C, C), lambda b, t: (0, 0)),
                  pl.BlockSpec((32, C), lambda b, t: (0, 0)),
                  pl.BlockSpec((1, C), lambda b, t: (0, 0)),
                  pl.BlockSpec((1, C), lambda b, t: (0, 0)),
                  pl.BlockSpec((4, C), lambda b, t: (0, 0))],
        out_specs=[pl.BlockSpec((None, 32, TN2), lambda b, t: (b, 0, t)),
                   pl.BlockSpec((None, 4, TN2), lambda b, t: (b, 0, t)),
                   pl.BlockSpec((32, 2), lambda b, t: (0, 0))],
        out_shape=[jax.ShapeDtypeStruct((B, 32, n), jnp.float32),
                   jax.ShapeDtypeStruct((B, 4, n), jnp.float32),
                   jax.ShapeDtypeStruct((32, 2), jnp.float32)],
    )(y1pre, st1, ut, w_pos, w_key, w_v1, w_q, w_k, w_attn)

    valpre, st3 = pl.pallas_call(
        _bn_mm_kernel,
        grid=g2,
        in_specs=[pl.BlockSpec((None, 32, TN2), lambda b, t: (b, 0, t)),
                  pl.BlockSpec((32, 2), lambda b, t: (0, 0)),
                  pl.BlockSpec((C, 32), lambda b, t: (0, 0))],
        out_specs=[xspec, stspec],
        out_shape=[jax.ShapeDtypeStruct((B, C, n), jnp.float32),
                   jax.ShapeDtypeStruct((C, 2), jnp.float32)],
    )(hpre, st2, w_v2)

    out = pl.pallas_call(
        _final_kernel,
        grid=(B,),
        in_specs=[pl.BlockSpec((None, C, n), lambda b: (b, 0, 0)),
                  pl.BlockSpec((C, 2), lambda b: (0, 0)),
                  pl.BlockSpec((None, 3, n), lambda b: (b, 0, 0)),
                  pl.BlockSpec((C, 3), lambda b: (0, 0)),
                  pl.BlockSpec((1, C), lambda b: (0, 0)),
                  pl.BlockSpec((None, 4, n), lambda b: (b, 0, 0))],
        out_specs=pl.BlockSpec((None, C, n), lambda b: (b, 0, 0)),
        out_shape=jax.ShapeDtypeStruct((B, C, n), jnp.float32),
    )(valpre, st3, ut, w_pos, w_vc, attnpre)
    return out
